# baseline (speedup 1.0000x reference)
import functools

import jax
import jax.numpy as jnp
from jax import lax
from jax.experimental import pallas as pl
from jax.experimental.pallas import tpu as pltpu
from jax.experimental.pallas import tpu_sc as plsc

B = 16384
D = 64
NC = 2
NS = 16
NW = NC * NS
ROWS = 2 * B
ROWS_PER_W = ROWS // NW

_mesh = plsc.VectorSubcoreMesh(core_axis_name="c", subcore_axis_name="s")


@functools.partial(
    pl.kernel,
    mesh=_mesh,
    out_type=jax.ShapeDtypeStruct((ROWS, D), jnp.float32),
    compiler_params=pltpu.CompilerParams(use_tc_tiling_on_sc=False),
    scratch_types=[
        pltpu.VMEM((ROWS_PER_W,), jnp.int32),
        pltpu.VMEM((ROWS_PER_W, D), jnp.float32),
        pltpu.SemaphoreType.DMA,
    ],
)
def _gather_rows(table_hbm, idx_hbm, out_hbm, idx_v, rows_v, sem):
    wid = lax.axis_index("s") * NC + lax.axis_index("c")
    base = wid * ROWS_PER_W
    pltpu.sync_copy(idx_hbm.at[pl.ds(base, ROWS_PER_W)], idx_v)
    pltpu.async_copy(table_hbm.at[idx_v], rows_v, sem).wait()
    pltpu.sync_copy(rows_v, out_hbm.at[pl.ds(base, ROWS_PER_W)])


def kernel(source_node_input, destination_node_input, embedding_table):
    idx = jnp.concatenate(
        [source_node_input, destination_node_input], axis=1
    ).reshape(ROWS)
    out = _gather_rows(embedding_table, idx)
    return out.reshape(B, 2 * D)


# speedup vs baseline: 1.6907x; 1.6907x over previous
import functools

import jax
import jax.numpy as jnp
from jax import lax
from jax.experimental import pallas as pl
from jax.experimental.pallas import tpu as pltpu
from jax.experimental.pallas import tpu_sc as plsc

B = 16384
D = 64
NC = 2
NS = 16
NW = NC * NS
EPW = B // NW
RPW = 2 * EPW
CH = 64
OPC = CH // 2
NCH = RPW // CH

_mesh = plsc.VectorSubcoreMesh(core_axis_name="c", subcore_axis_name="s")


@functools.partial(
    pl.kernel,
    mesh=_mesh,
    out_type=jax.ShapeDtypeStruct((B, 2 * D), jnp.float32),
    scratch_types=[
        pltpu.VMEM((EPW,), jnp.int32),
        pltpu.VMEM((EPW,), jnp.int32),
        pltpu.VMEM((CH, D), jnp.float32),
        pltpu.VMEM((CH, D), jnp.float32),
        pltpu.VMEM((OPC, 2 * D), jnp.float32),
        pltpu.SemaphoreType.DMA,
        pltpu.SemaphoreType.DMA,
    ],
)
def _edge_gather(src_hbm, dst_hbm, table_hbm, out_hbm,
                 srcv, dstv, row0, row1, ob, sem0, sem1):
    wid = lax.axis_index("s") * NC + lax.axis_index("c")
    eb = wid * EPW

    pltpu.sync_copy(src_hbm.at[pl.ds(eb, EPW)], srcv)
    pltpu.sync_copy(dst_hbm.at[pl.ds(eb, EPW)], dstv)

    def issue(c, buf, sem):
        for g in range(OPC // 16):
            vs = srcv[pl.ds(c * OPC + g * 16, 16)]
            vd = dstv[pl.ds(c * OPC + g * 16, 16)]
            for l in range(16):
                j = 2 * (g * 16 + l)
                pltpu.async_copy(
                    table_hbm.at[pl.ds(vs[l], 1), :],
                    buf.at[pl.ds(j, 1), :],
                    sem,
                )
                pltpu.async_copy(
                    table_hbm.at[pl.ds(vd[l], 1), :],
                    buf.at[pl.ds(j + 1, 1), :],
                    sem,
                )

    def drain(buf, sem):
        for j in range(CH):
            pltpu.make_async_copy(
                table_hbm.at[pl.ds(0, 1), :],
                buf.at[pl.ds(0, 1), :],
                sem,
            ).wait()

    def assemble_and_store(c, buf):
        for r in range(OPC):
            for j in range(D // 16):
                ob[r, pl.ds(16 * j, 16)] = buf[2 * r, pl.ds(16 * j, 16)]
                ob[r, pl.ds(D + 16 * j, 16)] = buf[2 * r + 1, pl.ds(16 * j, 16)]
        pltpu.sync_copy(ob, out_hbm.at[pl.ds(eb + c * OPC, OPC)])

    issue(0, row0, sem0)

    @pl.loop(0, NCH // 2)
    def pipeline(i):
        c0 = 2 * i
        issue(c0 + 1, row1, sem1)
        drain(row0, sem0)
        assemble_and_store(c0, row0)

        @pl.when(i < NCH // 2 - 1)
        def _():
            issue(c0 + 2, row0, sem0)

        drain(row1, sem1)
        assemble_and_store(c0 + 1, row1)


def kernel(source_node_input, destination_node_input, embedding_table):
    return _edge_gather(
        source_node_input.reshape(B),
        destination_node_input.reshape(B),
        embedding_table,
    )
